# Initial kernel scaffold; baseline (speedup 1.0000x reference)
import jax
import jax.numpy as jnp
from jax.experimental import pallas as pl


def kernel(x, edge_index, W_enc, b_enc, W_conv, b_conv, W_res, b_res, W_dec, b_dec, eps):
    raise NotImplementedError("write your pallas kernel here")


# speedup vs baseline: 18.6363x; 18.6363x over previous
import functools

import jax
import jax.numpy as jnp
from jax import lax
from jax.experimental import pallas as pl
from jax.experimental.pallas import tpu as pltpu
from jax.experimental.pallas import tpu_sc as plsc

N = 10000
E = 320000
NFEAT = 128
NHID = 128
NCLASS = 40
NLAYERS = 4
DT = 1.0
ALPHA = 1.0

NC = 2
NS = 16
NW = NC * NS
ECHUNK = 128
NPAD = 10240
EPAD = 327680
CH = EPAD // (NW * ECHUNK)
RPT = NPAD // NS
BLK = 256
GRID = NPAD // BLK

_MESH = plsc.VectorSubcoreMesh(
    core_axis_name="c", subcore_axis_name="s", num_cores=NC, num_subcores=NS)


@functools.partial(
    pl.kernel,
    out_type=jax.ShapeDtypeStruct((NC * NPAD,), jnp.float32),
    mesh=_MESH,
    scratch_types=[
        pltpu.VMEM((CH, ECHUNK), jnp.int32),
        pltpu.VMEM((ECHUNK,), jnp.float32),
        pltpu.VMEM_SHARED((NPAD,), jnp.float32),
    ],
)
def _degree_sc(dsts_hbm, zeros1_hbm, out_hbm, dst_v, ones_v, acc):
    c = lax.axis_index("c")
    s = lax.axis_index("s")
    wid = s * NC + c
    pltpu.sync_copy(zeros1_hbm.at[pl.ds(s * RPT, RPT)], acc.at[pl.ds(s * RPT, RPT)])
    pltpu.sync_copy(dsts_hbm.at[wid], dst_v)
    for i in range(ECHUNK // 16):
        ones_v[pl.ds(i * 16, 16)] = jnp.ones((16,), jnp.float32)
    plsc.subcore_barrier()

    def body(j, carry):
        pltpu.sync_copy(ones_v, acc.at[dst_v.at[j]], add=True)
        return carry

    lax.fori_loop(0, CH, body, 0)
    plsc.subcore_barrier()
    pltpu.sync_copy(acc.at[pl.ds(s * RPT, RPT)],
                    out_hbm.at[pl.ds(c * NPAD + s * RPT, RPT)])


@functools.partial(
    pl.kernel,
    out_type=jax.ShapeDtypeStruct((NC * NPAD, NHID), jnp.float32),
    mesh=_MESH,
    scratch_types=[
        pltpu.VMEM((CH, ECHUNK), jnp.int32),
        pltpu.VMEM((1, ECHUNK), jnp.int32),
        pltpu.VMEM((1, ECHUNK), jnp.int32),
        pltpu.VMEM((ECHUNK, NHID), jnp.float32),
        pltpu.VMEM((ECHUNK, NHID), jnp.float32),
        pltpu.VMEM_SHARED((NPAD, NHID), jnp.float32),
        pltpu.SemaphoreType.DMA,
        pltpu.SemaphoreType.DMA,
    ],
)
def _segsum_sc(y_hbm, srcs_hbm, dsts_hbm, zeros_hbm, out_hbm,
               dst_v, sidx_a, sidx_b, buf_a, buf_b, acc, sem_a, sem_b):
    c = lax.axis_index("c")
    s = lax.axis_index("s")
    wid = s * NC + c
    pltpu.sync_copy(zeros_hbm.at[pl.ds(s * RPT, RPT)], acc.at[pl.ds(s * RPT, RPT)])
    pltpu.sync_copy(dsts_hbm.at[wid], dst_v)
    pltpu.sync_copy(srcs_hbm.at[wid].at[pl.ds(0, 1)], sidx_a)
    plsc.subcore_barrier()

    pltpu.async_copy(y_hbm.at[sidx_a.at[0]], buf_a, sem_a)
    pltpu.sync_copy(srcs_hbm.at[wid].at[pl.ds(1, 1)], sidx_b)
    pltpu.async_copy(y_hbm.at[sidx_b.at[0]], buf_b, sem_b)

    def body(t, carry):
        j = 2 * t
        pltpu.make_async_copy(y_hbm.at[sidx_a.at[0]], buf_a, sem_a).wait()
        pltpu.sync_copy(buf_a, acc.at[dst_v.at[j]], add=True)

        @pl.when(j + 2 < CH)
        def _():
            pltpu.sync_copy(srcs_hbm.at[wid].at[pl.ds(j + 2, 1)], sidx_a)
            pltpu.async_copy(y_hbm.at[sidx_a.at[0]], buf_a, sem_a)

        pltpu.make_async_copy(y_hbm.at[sidx_b.at[0]], buf_b, sem_b).wait()
        pltpu.sync_copy(buf_b, acc.at[dst_v.at[j + 1]], add=True)

        @pl.when(j + 3 < CH)
        def _():
            pltpu.sync_copy(srcs_hbm.at[wid].at[pl.ds(j + 3, 1)], sidx_b)
            pltpu.async_copy(y_hbm.at[sidx_b.at[0]], buf_b, sem_b)

        return carry

    lax.fori_loop(0, CH // 2, body, 0)
    plsc.subcore_barrier()
    pltpu.sync_copy(acc.at[pl.ds(s * RPT, RPT)],
                    out_hbm.at[pl.ds(c * NPAD + s * RPT, RPT)])


def _row_spec(w):
    return pl.BlockSpec((BLK, w), lambda i: (i, 0))


def _full_spec(h, w):
    return pl.BlockSpec((h, w), lambda i: (0, 0))


def _enc_body(x_ref, wenc_ref, benc_ref, d0_ref, d1_ref, wc_ref, wres_ref,
              bcr_ref, x0_ref, y_ref, cc_ref, dinv_ref):
    deg = d0_ref[...] + d1_ref[...] + 1.0
    dinv = lax.rsqrt(deg)
    x = jnp.maximum(
        jnp.dot(x_ref[...], wenc_ref[...], preferred_element_type=jnp.float32)
        + benc_ref[...], 0.0)
    xw = jnp.dot(x, wc_ref[...], preferred_element_type=jnp.float32)
    x0_ref[...] = x
    y_ref[...] = dinv * xw
    cc_ref[...] = bcr_ref[...] - jnp.dot(xw, wres_ref[...],
                                         preferred_element_type=jnp.float32)
    dinv_ref[...] = dinv


_enc_tc = pl.pallas_call(
    _enc_body,
    grid=(GRID,),
    in_specs=[
        _row_spec(NFEAT),
        _full_spec(NFEAT, NHID),
        _full_spec(1, NHID),
        pl.BlockSpec((BLK, 1), lambda i: (i, 0)),
        pl.BlockSpec((BLK, 1), lambda i: (i, 0)),
        _full_spec(NHID, NHID),
        _full_spec(NHID, NHID),
        _full_spec(1, NHID),
    ],
    out_specs=[
        _row_spec(NHID), _row_spec(NHID), _row_spec(NHID),
        pl.BlockSpec((BLK, 1), lambda i: (i, 0)),
    ],
    out_shape=[
        jax.ShapeDtypeStruct((NPAD, NHID), jnp.float32),
        jax.ShapeDtypeStruct((NPAD, NHID), jnp.float32),
        jax.ShapeDtypeStruct((NPAD, NHID), jnp.float32),
        jax.ShapeDtypeStruct((NPAD, 1), jnp.float32),
    ],
)


def _layer_body(s0_ref, s1_ref, y_ref, c_ref, dinv_ref, x_ref, x0_ref,
                eps_ref, wc_ref, wres_ref, bcr_ref,
                x0o_ref, yo_ref, co_ref):
    coeff = 1.0 + jnp.tanh(eps_ref[...])
    dinv = dinv_ref[...]
    pre = dinv * (s0_ref[...] + s1_ref[...] + y_ref[...]) + c_ref[...]
    x0n = (x0_ref[...] * coeff
           + DT * (jnp.maximum(pre, 0.0) - ALPHA * x_ref[...]))
    xw = jnp.dot(x0n, wc_ref[...], preferred_element_type=jnp.float32)
    x0o_ref[...] = x0n
    yo_ref[...] = dinv * xw
    co_ref[...] = bcr_ref[...] - jnp.dot(xw, wres_ref[...],
                                         preferred_element_type=jnp.float32)


_layer_tc = pl.pallas_call(
    _layer_body,
    grid=(GRID,),
    in_specs=[
        _row_spec(NHID), _row_spec(NHID), _row_spec(NHID), _row_spec(NHID),
        pl.BlockSpec((BLK, 1), lambda i: (i, 0)),
        _row_spec(NHID), _row_spec(NHID),
        _full_spec(1, NHID),
        _full_spec(NHID, NHID),
        _full_spec(NHID, NHID),
        _full_spec(1, NHID),
    ],
    out_specs=[_row_spec(NHID), _row_spec(NHID), _row_spec(NHID)],
    out_shape=[
        jax.ShapeDtypeStruct((NPAD, NHID), jnp.float32),
        jax.ShapeDtypeStruct((NPAD, NHID), jnp.float32),
        jax.ShapeDtypeStruct((NPAD, NHID), jnp.float32),
    ],
)


def _final_body(s0_ref, s1_ref, y_ref, c_ref, dinv_ref, x_ref, x0_ref,
                eps_ref, wdec_ref, bdec_ref, out_ref):
    coeff = 1.0 + jnp.tanh(eps_ref[...])
    pre = dinv_ref[...] * (s0_ref[...] + s1_ref[...] + y_ref[...]) + c_ref[...]
    x0n = (x0_ref[...] * coeff
           + DT * (jnp.maximum(pre, 0.0) - ALPHA * x_ref[...]))
    out_ref[...] = jnp.dot(x0n, wdec_ref[...],
                           preferred_element_type=jnp.float32) + bdec_ref[...]


_final_tc = pl.pallas_call(
    _final_body,
    grid=(GRID,),
    in_specs=[
        _row_spec(NHID), _row_spec(NHID), _row_spec(NHID), _row_spec(NHID),
        pl.BlockSpec((BLK, 1), lambda i: (i, 0)),
        _row_spec(NHID), _row_spec(NHID),
        _full_spec(1, NHID),
        _full_spec(NHID, NCLASS),
        _full_spec(1, NCLASS),
    ],
    out_specs=[_row_spec(NCLASS)],
    out_shape=[jax.ShapeDtypeStruct((NPAD, NCLASS), jnp.float32)],
)


def kernel(x, edge_index, W_enc, b_enc, W_conv, b_conv, W_res, b_res,
           W_dec, b_dec, eps):
    f32 = jnp.float32
    src = edge_index[0]
    dst = edge_index[1]
    npad_extra = NPAD - N
    pad = EPAD - E
    ar = jnp.arange(pad, dtype=jnp.int32)
    src_p = jnp.concatenate([src, (ar * 37) % N]).reshape(NW, CH, ECHUNK)
    dst_p = jnp.concatenate([dst, N + (ar % npad_extra)]).reshape(NW, CH, ECHUNK)

    zeros1 = jnp.zeros((NPAD,), f32)
    zeros2 = jnp.zeros((NPAD, NHID), f32)
    x_pad = jnp.concatenate([x, jnp.zeros((npad_extra, NFEAT), f32)], axis=0)

    deg_parts = _degree_sc(dst_p, zeros1).reshape(NC, NPAD, 1)

    wencT = W_enc.T
    wcT = W_conv.T
    wresT = W_res.T
    wdecT = W_dec.T
    bcr = (b_conv - b_res).reshape(1, NHID)

    x0, y, cc, dinv = _enc_tc(
        x_pad, wencT, b_enc.reshape(1, NHID), deg_parts[0], deg_parts[1],
        wcT, wresT, bcr)
    xcur = x0

    for i in range(NLAYERS):
        s_parts = _segsum_sc(y, src_p, dst_p, zeros2).reshape(NC, NPAD, NHID)
        eps_i = eps[i].reshape(1, NHID)
        if i < NLAYERS - 1:
            x0, y, cc = _layer_tc(
                s_parts[0], s_parts[1], y, cc, dinv, xcur, x0,
                eps_i, wcT, wresT, bcr)
            xcur = x0
        else:
            (out,) = _final_tc(
                s_parts[0], s_parts[1], y, cc, dinv, xcur, x0,
                eps_i, wdecT, b_dec.reshape(1, NCLASS))
    return out[:N]
